# baseline (speedup 1.0000x reference)
import jax
import jax.numpy as jnp
from jax import lax
from jax.experimental import pallas as pl
from jax.experimental.pallas import tpu as pltpu
from jax.experimental.pallas import tpu_sc as plsc

WEEK = 7
DAY = 288
D = 64
LANES = 16
GATHER = 128
WINDOW = 512
GATHERS_PER_WINDOW = WINDOW // GATHER


def _build_combined_table(dow_table, tod_table):
    def body(dow_ref, tod_ref, c_ref):
        c_ref[...] = dow_ref[...][:, None, :] + tod_ref[...][None, :, :]

    return pl.pallas_call(
        body,
        out_shape=jax.ShapeDtypeStruct((WEEK, DAY, D), jnp.float32),
    )(dow_table, tod_table)


def _sc_gather(combined, dow_flat, tod_flat, n_rows):
    mesh = plsc.VectorSubcoreMesh(core_axis_name="c", subcore_axis_name="s")

    @pl.kernel(
        out_type=jax.ShapeDtypeStruct((n_rows, D), jnp.float32),
        mesh=mesh,
        scratch_types=[pltpu.VMEM((WINDOW,), jnp.int32), pltpu.SemaphoreType.DMA],
        compiler_params=pltpu.CompilerParams(use_tc_tiling_on_sc=False),
    )
    def k(c_hbm, dow_hbm, tod_hbm, out_hbm, idx_ref, sem):
        def body(dow_v, tod_v, out_v):
            @pl.loop(0, WINDOW, step=LANES)
            def _(i):
                sl = pl.ds(i, LANES)
                idx_ref[sl] = dow_v.at[0][sl] * DAY + tod_v.at[0][sl]

            copies = [
                pltpu.async_copy(
                    c_hbm.at[idx_ref.at[pl.ds(g * GATHER, GATHER)]],
                    out_v.at[pl.ds(g * GATHER, GATHER)],
                    sem,
                )
                for g in range(GATHERS_PER_WINDOW)
            ]
            for c in copies:
                c.wait()

        pltpu.emit_pipeline(
            body,
            grid=(n_rows // WINDOW,),
            in_specs=[
                pl.BlockSpec((1, WINDOW), index_map=lambda i: (0, i)),
                pl.BlockSpec((1, WINDOW), index_map=lambda i: (0, i)),
            ],
            out_specs=[pl.BlockSpec((WINDOW, D), index_map=lambda i: (i, 0))],
            core_axis_name=("c", "s"),
            dimension_semantics=(pltpu.PARALLEL,),
        )(dow_hbm, tod_hbm, out_hbm)

    return k(combined, dow_flat, tod_flat)


@jax.jit
def kernel(dow, tod, dow_table, tod_table):
    b, t = dow.shape
    n = b * t
    combined = _build_combined_table(dow_table, tod_table).reshape(WEEK * DAY, D)
    dow_flat = dow.reshape(1, n).astype(jnp.int32)
    tod_flat = tod.reshape(1, n).astype(jnp.int32)
    out = _sc_gather(combined, dow_flat, tod_flat, n)
    return out.reshape(b, t, D)


# speedup vs baseline: 1.7231x; 1.7231x over previous
import jax
import jax.numpy as jnp
from jax import lax
from jax.experimental import pallas as pl
from jax.experimental.pallas import tpu as pltpu
from jax.experimental.pallas import tpu_sc as plsc

WEEK = 7
DAY = 288
D = 64
LANES = 16
WINDOW = 128


def _build_combined_table(dow_table, tod_table):
    def body(dow_ref, tod_ref, c_ref):
        c_ref[...] = dow_ref[...][:, None, :] + tod_ref[...][None, :, :]

    return pl.pallas_call(
        body,
        out_shape=jax.ShapeDtypeStruct((WEEK, DAY, D), jnp.float32),
    )(dow_table, tod_table)


def _sc_gather(combined, dow_flat, tod_flat, n_rows):
    mesh = plsc.VectorSubcoreMesh(core_axis_name="c", subcore_axis_name="s")

    @pl.kernel(
        out_type=jax.ShapeDtypeStruct((n_rows, D), jnp.float32),
        mesh=mesh,
        scratch_types=[
            pltpu.VMEM((WINDOW,), jnp.int32),
            pltpu.VMEM_SHARED((WEEK * DAY, D), jnp.float32),
        ],
        compiler_params=pltpu.CompilerParams(use_tc_tiling_on_sc=False),
    )
    def k(c_hbm, dow_hbm, tod_hbm, out_hbm, idx_ref, c_shared):
        @pl.when(lax.axis_index("s") == 0)
        def _():
            pltpu.sync_copy(c_hbm, c_shared)

        plsc.subcore_barrier()

        def body(dow_v, tod_v, out_v):
            @pl.loop(0, WINDOW, step=LANES)
            def _(i):
                sl = pl.ds(i, LANES)
                idx_ref[sl] = dow_v.at[0][sl] * DAY + tod_v.at[0][sl]

            pltpu.sync_copy(c_shared.at[idx_ref], out_v)

        pltpu.emit_pipeline(
            body,
            grid=(n_rows // WINDOW,),
            in_specs=[
                pl.BlockSpec((1, WINDOW), index_map=lambda i: (0, i)),
                pl.BlockSpec((1, WINDOW), index_map=lambda i: (0, i)),
            ],
            out_specs=[pl.BlockSpec((WINDOW, D), index_map=lambda i: (i, 0))],
            core_axis_name=("c", "s"),
            dimension_semantics=(pltpu.PARALLEL,),
        )(dow_hbm, tod_hbm, out_hbm)

    return k(combined, dow_flat, tod_flat)


@jax.jit
def kernel(dow, tod, dow_table, tod_table):
    b, t = dow.shape
    n = b * t
    combined = _build_combined_table(dow_table, tod_table).reshape(WEEK * DAY, D)
    dow_flat = dow.reshape(1, n).astype(jnp.int32)
    tod_flat = tod.reshape(1, n).astype(jnp.int32)
    out = _sc_gather(combined, dow_flat, tod_flat, n)
    return out.reshape(b, t, D)
